# baseline (speedup 1.0000x reference)
import jax
import jax.numpy as jnp
from jax.experimental import pallas as pl


def kernel(input_features, neighbor_inds, guidance, weightnet):
    b, n, c = input_features.shape
    k = neighbor_inds.shape[2]
    h = guidance.shape[3]
    c_mid = weightnet.shape[3]
    gathered = jax.vmap(lambda feat, inds: jnp.take(feat, inds, axis=0))(
        input_features, neighbor_inds
    )
    guided = gathered.reshape(b, n, k, h, c // h) * guidance[..., None]
    guided = guided.reshape(b, n, k, c)
    out = jnp.einsum("bnkc,bnkm->bncm", guided, weightnet)
    return out.reshape(b, n, c * c_mid)


# speedup vs baseline: 9.5796x; 9.5796x over previous
import functools

import jax
import jax.numpy as jnp
from jax import lax
from jax.experimental import pallas as pl
from jax.experimental.pallas import tpu as pltpu
from jax.experimental.pallas import tpu_sc as plsc

_N = 100000
_K = 16
_C = 32
_H = 8
_M = 16
_NW = 32
_P = 25
_CHUNKS = _N // (_NW * _P)
_ROWS = _P * _K
_GSPLIT = 4
_RPG = _ROWS // _GSPLIT
_GD_C = _P * _K * _H
_W_C = _P * _K * _M
_OUT_C = _P * _C * _M


def _dyn_gather(v, idx):
    return lax.gather(
        v, idx[:, None],
        lax.GatherDimensionNumbers(
            offset_dims=(), collapsed_slice_dims=(0,), start_index_map=(0,)),
        slice_sizes=(1,),
        mode=lax.GatherScatterMode.PROMISE_IN_BOUNDS)


def _pcf_body(feat_hbm, inds_hbm, gd_hbm, w_hbm, out_hbm,
              idx_v, rows_v, gd_v, w_v, out_v, gsem, lsem, osem):
    wid = lax.axis_index("s") * 2 + lax.axis_index("c")
    g0 = wid * _CHUNKS
    iota = lax.iota(jnp.int32, 16)
    head_lo = lax.shift_right_logical(iota, 2)
    zero16 = iota * 0
    bc_idx = [zero16 + c for c in range(16)]

    def start_loads(g, slot):
        pltpu.sync_copy(inds_hbm.at[g], idx_v.at[slot])
        for j in range(_GSPLIT):
            pltpu.async_copy(feat_hbm.at[idx_v.at[slot, j]],
                             rows_v.at[slot, pl.ds(j * _RPG, _RPG)],
                             gsem.at[slot])
        pltpu.async_copy(gd_hbm.at[pl.ds(g * _GD_C, _GD_C)],
                         gd_v.at[slot], lsem.at[slot])
        pltpu.async_copy(w_hbm.at[pl.ds(g * _W_C, _W_C)],
                         w_v.at[slot], lsem.at[slot])

    def wait_loads(g, slot):
        for j in range(_GSPLIT):
            pltpu.make_async_copy(feat_hbm.at[idx_v.at[slot, j]],
                                  rows_v.at[slot, pl.ds(j * _RPG, _RPG)],
                                  gsem.at[slot]).wait()
        pltpu.make_async_copy(gd_hbm.at[pl.ds(g * _GD_C, _GD_C)],
                              gd_v.at[slot], lsem.at[slot]).wait()
        pltpu.make_async_copy(w_hbm.at[pl.ds(g * _W_C, _W_C)],
                              w_v.at[slot], lsem.at[slot]).wait()

    def out_copy(g, slot):
        return pltpu.make_async_copy(
            out_v.at[slot], out_hbm.at[pl.ds(g * _OUT_C, _OUT_C)],
            osem.at[slot])

    start_loads(g0, 0)

    def chunk_body(i, carry):
        b = lax.rem(i, 2)
        nb = 1 - b
        g = g0 + i

        @pl.when(i < _CHUNKS - 1)
        def _prefetch():
            start_loads(g + 1, nb)

        wait_loads(g, b)

        def point_body(p, pc):
            acc = [jnp.zeros((16,), jnp.float32) for _ in range(_C)]
            gdp = None
            for kk in range(_K):
                if kk % 2 == 0:
                    gdp = gd_v[b, pl.ds(p * (_K * _H) + kk * _H, 16)]
                    off = 0
                else:
                    off = 8
                e0 = _dyn_gather(gdp, head_lo + off)
                e1 = _dyn_gather(gdp, head_lo + (off + 4))
                f0 = rows_v[b, p * _K + kk, pl.ds(0, 16)]
                f1 = rows_v[b, p * _K + kk, pl.ds(16, 16)]
                gm0 = f0 * e0
                gm1 = f1 * e1
                wk = w_v[b, pl.ds(p * (_K * _M) + kk * _M, 16)]
                for c in range(16):
                    acc[c] = acc[c] + _dyn_gather(gm0, bc_idx[c]) * wk
                    acc[16 + c] = acc[16 + c] + _dyn_gather(gm1, bc_idx[c]) * wk
            for c in range(_C):
                out_v[b, pl.ds(p * (_C * _M) + c * 16, 16)] = acc[c]
            return pc

        lax.fori_loop(0, _P, point_body, 0)

        out_copy(g, b).start()

        @pl.when(i >= 1)
        def _drain_prev_store():
            out_copy(g - 1, nb).wait()

        return carry

    lax.fori_loop(0, _CHUNKS, chunk_body, 0)
    out_copy(g0 + _CHUNKS - 1, (_CHUNKS - 1) % 2).wait()


@jax.jit
def _pcf_call(feat, inds, gd, w):
    mesh = plsc.VectorSubcoreMesh(core_axis_name="c", subcore_axis_name="s")
    kfn = functools.partial(
        pl.kernel,
        mesh=mesh,
        compiler_params=pltpu.CompilerParams(use_tc_tiling_on_sc=False),
        out_type=jax.ShapeDtypeStruct((_N * _C * _M,), jnp.float32),
        scratch_types=[
            pltpu.VMEM((2, _GSPLIT, _RPG), jnp.int32),
            pltpu.VMEM((2, _ROWS, _C), jnp.float32),
            pltpu.VMEM((2, _GD_C), jnp.float32),
            pltpu.VMEM((2, _W_C), jnp.float32),
            pltpu.VMEM((2, _OUT_C), jnp.float32),
            pltpu.SemaphoreType.DMA((2,)),
            pltpu.SemaphoreType.DMA((2,)),
            pltpu.SemaphoreType.DMA((2,)),
        ],
    )(_pcf_body)
    return kfn(feat, inds, gd, w)


def kernel(input_features, neighbor_inds, guidance, weightnet):
    b, n, c = input_features.shape
    k = neighbor_inds.shape[2]
    h = guidance.shape[3]
    m = weightnet.shape[3]
    assert (b, n, c, k, h, m) == (1, _N, _C, _K, _H, _M)
    feat = input_features.reshape(n, c)
    inds = neighbor_inds.astype(jnp.int32).reshape(
        n * k // (_GSPLIT * _RPG), _GSPLIT, _RPG)
    gd = guidance.reshape(n * k * h)
    w = weightnet.reshape(n * k * m)
    out = _pcf_call(feat, inds, gd, w)
    return out.reshape(b, n, c * m)
